# Initial kernel scaffold; baseline (speedup 1.0000x reference)
import functools

import jax
import jax.numpy as jnp
from jax import lax
from jax.experimental import pallas as pl
from jax.experimental.pallas import tpu as pltpu
from jax.experimental.pallas import tpu_sc as plsc

N = 10000
D = 128
NP = 10016
RPS = NP // 16
CHUNK = 128
NTILES = 32
CW = 16


def _cdiv(a, b):
    return (a + b - 1) // b


def _make_seg_sum(chunks_per_tile, with_counts):
    mesh = plsc.VectorSubcoreMesh(core_axis_name="c", subcore_axis_name="s")
    out_type = [jax.ShapeDtypeStruct((2, NP, D), jnp.float32)]
    if with_counts:
        out_type.append(jax.ShapeDtypeStruct((2, NP, CW), jnp.float32))
    scratch = [
        pltpu.VMEM((CHUNK,), jnp.int32),
        pltpu.VMEM((CHUNK,), jnp.int32),
        pltpu.VMEM((CHUNK, D), jnp.float32),
        pltpu.VMEM_SHARED((NP, D), jnp.float32),
        pltpu.SemaphoreType.DMA,
    ]
    if with_counts:
        scratch += [
            pltpu.VMEM((CHUNK, CW), jnp.float32),
            pltpu.VMEM_SHARED((NP, CW), jnp.float32),
        ]

    def body(*refs):
        if with_counts:
            (table, src, dst, z128, zcw, out, cnt_out,
             sidx, didx, rows, acc, sem, ones, cacc) = refs
        else:
            (table, src, dst, z128, out,
             sidx, didx, rows, acc, sem) = refs
        c = lax.axis_index("c")
        s = lax.axis_index("s")
        wid = s * 2 + c

        r0 = s * RPS
        pltpu.sync_copy(z128.at[pl.ds(r0, RPS)], acc.at[pl.ds(r0, RPS)])
        if with_counts:
            pltpu.sync_copy(zcw.at[pl.ds(r0, RPS)], cacc.at[pl.ds(r0, RPS)])

            def init_ones(r, _):
                ones[r, :] = jnp.ones((CW,), jnp.float32)
                return 0
            lax.fori_loop(0, CHUNK, init_ones, 0)
        plsc.subcore_barrier()

        def step(j, _):
            eoff = (wid * chunks_per_tile + j) * CHUNK
            pltpu.sync_copy(src.at[pl.ds(eoff, CHUNK)], sidx)
            pltpu.sync_copy(dst.at[pl.ds(eoff, CHUNK)], didx)
            pltpu.async_copy(table.at[sidx], rows, sem).wait()
            pltpu.sync_copy(rows, acc.at[didx], add=True)
            if with_counts:
                pltpu.sync_copy(ones, cacc.at[didx], add=True)
            return 0
        lax.fori_loop(0, chunks_per_tile, step, 0)

        plsc.subcore_barrier()
        pltpu.sync_copy(acc.at[pl.ds(r0, RPS)], out.at[c, pl.ds(r0, RPS)])
        if with_counts:
            pltpu.sync_copy(cacc.at[pl.ds(r0, RPS)],
                            cnt_out.at[c, pl.ds(r0, RPS)])

    return pl.kernel(body, out_type=out_type, mesh=mesh, scratch_types=scratch)


def _pad_edges(ei, chunks_per_tile):
    total = chunks_per_tile * NTILES * CHUNK
    pad = total - ei.shape[1]
    src = jnp.concatenate([ei[0], jnp.zeros((pad,), jnp.int32)])
    dst = jnp.concatenate([ei[1], jnp.full((pad,), N, jnp.int32)])
    return src, dst


_ROWS = 1000


def _mm_t(a, w):
    return lax.dot_general(a, w, (((1,), (1,)), ((), ())),
                           preferred_element_type=jnp.float32)


def _sage_body(p_ref, c_ref, x_ref, wl_ref, bl_ref, wr_ref, o_ref):
    ssum = p_ref[0] + p_ref[1]
    cnt = c_ref[0, :, 0:1] + c_ref[1, :, 0:1]
    m = ssum / jnp.maximum(cnt, 1.0)
    o = _mm_t(m, wl_ref[...]) + bl_ref[...] + _mm_t(x_ref[...], wr_ref[...])
    nrm = jnp.sqrt(jnp.sum(o * o, axis=-1, keepdims=True))
    o = o / jnp.maximum(nrm, 1e-12)
    o_ref[...] = jnp.maximum(o, 0.0)


def _head_body(p_ref, c_ref, wlin_ref, blin_ref, o_ref):
    ssum = p_ref[0] + p_ref[1]
    cnt = c_ref[0, :, 0:1] + c_ref[1, :, 0:1]
    m = ssum / jnp.maximum(cnt, 1.0)
    o_ref[...] = _mm_t(m, wlin_ref[...]) + blin_ref[...]


def _sage_tc(p, cnt, x, Wl, bl, Wr):
    grid = (N // _ROWS,)
    return pl.pallas_call(
        _sage_body,
        grid=grid,
        in_specs=[
            pl.BlockSpec((2, _ROWS, D), lambda i: (0, i, 0)),
            pl.BlockSpec((2, _ROWS, CW), lambda i: (0, i, 0)),
            pl.BlockSpec((_ROWS, D), lambda i: (i, 0)),
            pl.BlockSpec((D, D), lambda i: (0, 0)),
            pl.BlockSpec((1, D), lambda i: (0, 0)),
            pl.BlockSpec((D, D), lambda i: (0, 0)),
        ],
        out_specs=pl.BlockSpec((_ROWS, D), lambda i: (i, 0)),
        out_shape=jax.ShapeDtypeStruct((N, D), jnp.float32),
    )(p, cnt, x, Wl, bl.reshape(1, D), Wr)


def _head_tc(p, cnt, Wlin, blin):
    grid = (N // _ROWS,)
    return pl.pallas_call(
        _head_body,
        grid=grid,
        in_specs=[
            pl.BlockSpec((2, _ROWS, D), lambda i: (0, i, 0)),
            pl.BlockSpec((2, _ROWS, CW), lambda i: (0, i, 0)),
            pl.BlockSpec((D, D), lambda i: (0, 0)),
            pl.BlockSpec((1, D), lambda i: (0, 0)),
        ],
        out_specs=pl.BlockSpec((_ROWS, D), lambda i: (i, 0)),
        out_shape=jax.ShapeDtypeStruct((N, D), jnp.float32),
    )(p, cnt, Wlin, blin.reshape(1, D))


def kernel(x, edge_index, k_hop_edge_index, Wl1, bl1, Wr1, Wl2, bl2, Wr2,
           Wlin, blin):
    cpt_e = _cdiv(edge_index.shape[1], NTILES * CHUNK)
    cpt_k = _cdiv(k_hop_edge_index.shape[1], NTILES * CHUNK)
    srcE, dstE = _pad_edges(edge_index, cpt_e)
    srcK, dstK = _pad_edges(k_hop_edge_index, cpt_k)
    z128 = jnp.zeros((NP, D), jnp.float32)
    zcw = jnp.zeros((NP, CW), jnp.float32)

    seg_c = _make_seg_sum(cpt_e, True)
    seg = _make_seg_sum(cpt_e, False)
    seg_k = _make_seg_sum(cpt_k, True)

    p1, c1 = seg_c(x, srcE, dstE, z128, zcw)
    h1 = _sage_tc(p1, c1, x, Wl1, bl1, Wr1)
    p2 = seg(h1, srcE, dstE, z128)
    h2 = _sage_tc(p2, c1, h1, Wl2, bl2, Wr2)
    p3, c3 = seg_k(h2, srcK, dstK, z128, zcw)
    return _head_tc(p3, c3, Wlin, blin)


# speedup vs baseline: 3.1695x; 3.1695x over previous
import jax
import jax.numpy as jnp
from jax import lax
from jax.experimental import pallas as pl
from jax.experimental.pallas import tpu as pltpu
from jax.experimental.pallas import tpu_sc as plsc

N = 10000
D = 128
NP = 10112
RPS = NP // 16
CHUNK = 128
NTILES = 32


def _cdiv(a, b):
    return (a + b - 1) // b


def _make_seg_sum(chunks_per_tile):
    mesh = plsc.VectorSubcoreMesh(core_axis_name="c", subcore_axis_name="s")

    def body(table, src, dst, z128, out, sidx, didx, rows, acc, sem):
        c = lax.axis_index("c")
        s = lax.axis_index("s")
        wid = s * 2 + c
        r0 = s * RPS
        pltpu.sync_copy(z128.at[pl.ds(r0, RPS)], acc.at[pl.ds(r0, RPS)])
        plsc.subcore_barrier()

        def step(j, _):
            eoff = (wid * chunks_per_tile + j) * CHUNK
            pltpu.sync_copy(src.at[pl.ds(eoff, CHUNK)], sidx)
            pltpu.sync_copy(dst.at[pl.ds(eoff, CHUNK)], didx)
            pltpu.async_copy(table.at[sidx], rows, sem).wait()
            pltpu.sync_copy(rows, acc.at[didx], add=True)
            return 0
        lax.fori_loop(0, chunks_per_tile, step, 0)

        plsc.subcore_barrier()
        pltpu.sync_copy(acc.at[pl.ds(r0, RPS)], out.at[c, pl.ds(r0, RPS)])

    return pl.kernel(
        body,
        out_type=[jax.ShapeDtypeStruct((2, NP, D), jnp.float32)],
        mesh=mesh,
        scratch_types=[
            pltpu.VMEM((CHUNK,), jnp.int32),
            pltpu.VMEM((CHUNK,), jnp.int32),
            pltpu.VMEM((CHUNK, D), jnp.float32),
            pltpu.VMEM_SHARED((NP, D), jnp.float32),
            pltpu.SemaphoreType.DMA,
        ],
    )


def _make_counts(cpt_e, cpt_k):
    mesh = plsc.VectorSubcoreMesh(core_axis_name="c", subcore_axis_name="s")

    def body(dstE, dstK, z128, o128, ce_out, ck_out, didx, ones, acc):
        c = lax.axis_index("c")
        s = lax.axis_index("s")
        wid = s * 2 + c
        r0 = s * RPS
        pltpu.sync_copy(o128.at[pl.ds(0, CHUNK)], ones)

        for dst, cpt, out in ((dstE, cpt_e, ce_out), (dstK, cpt_k, ck_out)):
            pltpu.sync_copy(z128.at[pl.ds(r0, RPS)], acc.at[pl.ds(r0, RPS)])
            plsc.subcore_barrier()

            def step(j, _, dst=dst, cpt=cpt):
                eoff = (wid * cpt + j) * CHUNK
                pltpu.sync_copy(dst.at[pl.ds(eoff, CHUNK)], didx)
                pltpu.sync_copy(ones, acc.at[didx], add=True)
                return 0
            lax.fori_loop(0, cpt, step, 0)

            plsc.subcore_barrier()
            pltpu.sync_copy(acc.at[pl.ds(r0, RPS)], out.at[c, pl.ds(r0, RPS)])

    return pl.kernel(
        body,
        out_type=[jax.ShapeDtypeStruct((2, NP, D), jnp.float32),
                  jax.ShapeDtypeStruct((2, NP, D), jnp.float32)],
        mesh=mesh,
        scratch_types=[
            pltpu.VMEM((CHUNK,), jnp.int32),
            pltpu.VMEM((CHUNK, D), jnp.float32),
            pltpu.VMEM_SHARED((NP, D), jnp.float32),
        ],
    )


def _pad_edges(ei, chunks_per_tile):
    total = chunks_per_tile * NTILES * CHUNK
    pad = total - ei.shape[1]
    src = jnp.concatenate([ei[0], jnp.zeros((pad,), jnp.int32)])
    dst = jnp.concatenate([ei[1], jnp.full((pad,), N, jnp.int32)])
    return src, dst


_ROWS = 1000


def _mm_t(a, w):
    return lax.dot_general(a, w, (((1,), (1,)), ((), ())),
                           preferred_element_type=jnp.float32)


def _sage_body(p_ref, c_ref, x_ref, wl_ref, bl_ref, wr_ref, o_ref):
    ssum = p_ref[0] + p_ref[1]
    cnt = c_ref[0, :, 0:1] + c_ref[1, :, 0:1]
    m = ssum / jnp.maximum(cnt, 1.0)
    o = _mm_t(m, wl_ref[...]) + bl_ref[...] + _mm_t(x_ref[...], wr_ref[...])
    nrm = jnp.sqrt(jnp.sum(o * o, axis=-1, keepdims=True))
    o = o / jnp.maximum(nrm, 1e-12)
    o_ref[...] = jnp.maximum(o, 0.0)


def _head_body(p_ref, c_ref, wlin_ref, blin_ref, o_ref):
    ssum = p_ref[0] + p_ref[1]
    cnt = c_ref[0, :, 0:1] + c_ref[1, :, 0:1]
    m = ssum / jnp.maximum(cnt, 1.0)
    o_ref[...] = _mm_t(m, wlin_ref[...]) + blin_ref[...]


def _sage_tc(p, cnt, x, Wl, bl, Wr):
    grid = (N // _ROWS,)
    return pl.pallas_call(
        _sage_body,
        grid=grid,
        in_specs=[
            pl.BlockSpec((2, _ROWS, D), lambda i: (0, i, 0)),
            pl.BlockSpec((2, _ROWS, D), lambda i: (0, i, 0)),
            pl.BlockSpec((_ROWS, D), lambda i: (i, 0)),
            pl.BlockSpec((D, D), lambda i: (0, 0)),
            pl.BlockSpec((1, D), lambda i: (0, 0)),
            pl.BlockSpec((D, D), lambda i: (0, 0)),
        ],
        out_specs=pl.BlockSpec((_ROWS, D), lambda i: (i, 0)),
        out_shape=jax.ShapeDtypeStruct((N, D), jnp.float32),
    )(p, cnt, x, Wl, bl.reshape(1, D), Wr)


def _head_tc(p, cnt, Wlin, blin):
    grid = (N // _ROWS,)
    return pl.pallas_call(
        _head_body,
        grid=grid,
        in_specs=[
            pl.BlockSpec((2, _ROWS, D), lambda i: (0, i, 0)),
            pl.BlockSpec((2, _ROWS, D), lambda i: (0, i, 0)),
            pl.BlockSpec((D, D), lambda i: (0, 0)),
            pl.BlockSpec((1, D), lambda i: (0, 0)),
        ],
        out_specs=pl.BlockSpec((_ROWS, D), lambda i: (i, 0)),
        out_shape=jax.ShapeDtypeStruct((N, D), jnp.float32),
    )(p, cnt, Wlin, blin.reshape(1, D))


def kernel(x, edge_index, k_hop_edge_index, Wl1, bl1, Wr1, Wl2, bl2, Wr2,
           Wlin, blin):
    cpt_e = _cdiv(edge_index.shape[1], NTILES * CHUNK)
    cpt_k = _cdiv(k_hop_edge_index.shape[1], NTILES * CHUNK)
    srcE, dstE = _pad_edges(edge_index, cpt_e)
    srcK, dstK = _pad_edges(k_hop_edge_index, cpt_k)
    z128 = jnp.zeros((NP, D), jnp.float32)
    o128 = jnp.ones((CHUNK, D), jnp.float32)

    seg_e = _make_seg_sum(cpt_e)
    seg_k = _make_seg_sum(cpt_k)
    counts = _make_counts(cpt_e, cpt_k)

    ce, ck = counts(dstE, dstK, z128, o128)
    (p1,) = seg_e(x, srcE, dstE, z128)
    h1 = _sage_tc(p1, ce, x, Wl1, bl1, Wr1)
    (p2,) = seg_e(h1, srcE, dstE, z128)
    h2 = _sage_tc(p2, ce, h1, Wl2, bl2, Wr2)
    (p3,) = seg_k(h2, srcK, dstK, z128)
    return _head_tc(p3, ck, Wlin, blin)
